# baseline (speedup 1.0000x reference)
import jax
import jax.numpy as jnp
from jax import lax
from jax.experimental import pallas as pl
from jax.experimental.pallas import tpu as pltpu, tpu_sc as plsc

N_NODES = 10000
FEAT = 1433
D = 512
B_ROWS = 1024
S = 48
U_SIZE = 16384
T = B_ROWS * S

NC, NS = 2, 16
NW = NC * NS
TPW = T // NW
BPW = B_ROWS // NW
CHUNK = 64
NCH = TPW // CHUNK

BM = 2560


def _proj_body(ft_ref, w_ref, b_ref, o_ref):
    o_ref[...] = (
        lax.dot_general(ft_ref[...].astype(jnp.bfloat16),
                        w_ref[...].astype(jnp.bfloat16),
                        (((0,), (0,)), ((), ())),
                        preferred_element_type=jnp.float32)
        + b_ref[...]
    )


def _mask_body(seq_ref, o_ref):
    col = lax.broadcasted_iota(jnp.int32, (B_ROWS, S), 1) + 1
    o_ref[...] = col > seq_ref[...]


def _sc_gather_body(proj_hbm, uni_hbm, samp_hbm, nodes_hbm,
                    neighs_out, nodes_out,
                    uni_v, samp_v, nid_v, nidx_v, nrows_v, rows0, rows1,
                    gsem0, gsem1, osem0, osem1, nsem):
    sid = lax.axis_index("s")
    wid = lax.axis_index("c") * NS + sid
    base = wid * TPW
    nb = wid * BPW

    uni_cp = pltpu.async_copy(uni_hbm, uni_v, gsem0)
    samp_cp = pltpu.async_copy(samp_hbm.at[pl.ds(base, TPW)], samp_v, gsem1)
    nidx_cp = pltpu.async_copy(nodes_hbm.at[pl.ds(nb, BPW)], nidx_v, nsem)
    uni_cp.wait()
    samp_cp.wait()
    nidx_cp.wait()

    gsems = (gsem0, gsem1)
    osems = (osem0, osem1)

    def compute_nid(c):
        for j in range(CHUNK // 16):
            idx = samp_v[pl.ds(c * CHUNK + j * 16, 16)]
            nid_v[c, pl.ds(j * 16, 16)] = plsc.load_gather(uni_v, [idx])

    gcp = [None, None]
    ocp = [None, None]
    compute_nid(0)
    gcp[0] = pltpu.async_copy(proj_hbm.at[nid_v.at[0]], rows0, gsems[0])
    node_cp = pltpu.async_copy(proj_hbm.at[nidx_v], nrows_v, nsem)
    rows = (rows0, rows1)
    for c in range(NCH):
        cur = c % 2
        nxt = 1 - cur
        if c + 1 < NCH:
            compute_nid(c + 1)
            if c >= 1:
                ocp[nxt].wait()
            gcp[nxt] = pltpu.async_copy(
                proj_hbm.at[nid_v.at[c + 1]], rows[nxt], gsems[nxt])
        gcp[cur].wait()
        ocp[cur] = pltpu.async_copy(
            rows[cur], neighs_out.at[pl.ds(base + c * CHUNK, CHUNK)],
            osems[cur])
    ocp[0].wait()
    ocp[1].wait()

    node_cp.wait()
    pltpu.sync_copy(nrows_v, nodes_out.at[pl.ds(nb, BPW)])


_sc_gather = pl.kernel(
    _sc_gather_body,
    out_type=[
        jax.ShapeDtypeStruct((T, D), jnp.float32),
        jax.ShapeDtypeStruct((B_ROWS, D), jnp.float32),
    ],
    mesh=plsc.VectorSubcoreMesh(core_axis_name="c", subcore_axis_name="s"),
    compiler_params=pltpu.CompilerParams(needs_layout_passes=False),
    scratch_types=[
        pltpu.VMEM((U_SIZE,), jnp.int32),
        pltpu.VMEM((TPW,), jnp.int32),
        pltpu.VMEM((NCH, CHUNK), jnp.int32),
        pltpu.VMEM((BPW,), jnp.int32),
        pltpu.VMEM((BPW, D), jnp.float32),
        pltpu.VMEM((CHUNK, D), jnp.float32),
        pltpu.VMEM((CHUNK, D), jnp.float32),
        pltpu.SemaphoreType.DMA,
        pltpu.SemaphoreType.DMA,
        pltpu.SemaphoreType.DMA,
        pltpu.SemaphoreType.DMA,
        pltpu.SemaphoreType.DMA,
    ],
)


def kernel(nodes, unique_nodes_list, samp_neighs_t, seq_length, features,
           W_in, b_in):
    nodes_i = nodes.astype(jnp.int32)
    uni_i = unique_nodes_list.astype(jnp.int32)
    samp_i = samp_neighs_t.astype(jnp.int32).reshape(-1)

    proj = pl.pallas_call(
        _proj_body,
        grid=(pl.cdiv(N_NODES, BM),),
        in_specs=[
            pl.BlockSpec((FEAT, BM), lambda i: (0, i)),
            pl.BlockSpec((FEAT, D), lambda i: (0, 0)),
            pl.BlockSpec((1, D), lambda i: (0, 0)),
        ],
        out_specs=pl.BlockSpec((BM, D), lambda i: (i, 0)),
        out_shape=jax.ShapeDtypeStruct((N_NODES, D), jnp.float32),
        compiler_params=pltpu.CompilerParams(
            fuse_transposed_lhs_in_matmul=True),
    )(features.T, W_in, b_in.reshape(1, D))

    mask = pl.pallas_call(
        _mask_body,
        out_shape=jax.ShapeDtypeStruct((B_ROWS, S), jnp.bool_),
    )(seq_length.astype(jnp.int32).reshape(B_ROWS, 1))

    neighs_flat, nodes_emb = _sc_gather(proj, uni_i, samp_i, nodes_i)
    return (nodes_emb, neighs_flat.reshape(B_ROWS, S, D), samp_neighs_t,
            mask)


# speedup vs baseline: 1.0077x; 1.0077x over previous
import jax
import jax.numpy as jnp
from jax import lax
from jax.experimental import pallas as pl
from jax.experimental.pallas import tpu as pltpu, tpu_sc as plsc

N_NODES = 10000
FEAT = 1433
D = 512
B_ROWS = 1024
S = 48
U_SIZE = 16384
T = B_ROWS * S

NC, NS = 2, 16
NW = NC * NS
TPW = T // NW
BPW = B_ROWS // NW
CHUNK = 64
NCH = TPW // CHUNK

BM = 2048


def _proj_body(ft_ref, w_ref, b_ref, o_ref):
    o_ref[...] = (
        lax.dot_general(ft_ref[...].astype(jnp.bfloat16),
                        w_ref[...].astype(jnp.bfloat16),
                        (((0,), (0,)), ((), ())),
                        preferred_element_type=jnp.float32)
        + b_ref[...]
    )


def _mask_body(seq_ref, o_ref):
    col = lax.broadcasted_iota(jnp.int32, (B_ROWS, S), 1) + 1
    o_ref[...] = col > seq_ref[...]


def _sc_gather_body(proj_hbm, uni_hbm, samp_hbm, nodes_hbm,
                    neighs_out, nodes_out,
                    uni_v, samp_v, nid_v, nidx_v, rows0, rows1, rows2,
                    gsem0, gsem1, gsem2, osem0, osem1, osem2, nsem):
    sid = lax.axis_index("s")
    wid = lax.axis_index("c") * NS + sid
    base = wid * TPW
    nb = wid * BPW

    uni_cp = pltpu.async_copy(uni_hbm, uni_v, gsem0)
    samp_cp = pltpu.async_copy(samp_hbm.at[pl.ds(base, TPW)], samp_v, gsem1)
    nidx_cp = pltpu.async_copy(nodes_hbm.at[pl.ds(nb, BPW)], nidx_v, nsem)
    uni_cp.wait()
    samp_cp.wait()
    nidx_cp.wait()

    gsems = (gsem0, gsem1, gsem2)
    osems = (osem0, osem1, osem2)
    rows = (rows0, rows1, rows2)
    NB = 3

    def compute_nid(c):
        for j in range(CHUNK // 16):
            idx = samp_v[pl.ds(c * CHUNK + j * 16, 16)]
            nid_v[c, pl.ds(j * 16, 16)] = plsc.load_gather(uni_v, [idx])

    def gather(c):
        return pltpu.async_copy(proj_hbm.at[nid_v.at[c]], rows[c % NB],
                                gsems[c % NB])

    gcp = [None, None, None]
    ocp = [None, None, None]
    compute_nid(0)
    gcp[0] = gather(0)
    compute_nid(1)
    gcp[1] = gather(1)
    for c in range(NCH):
        cur = c % NB
        nxt = c + NB - 1
        if nxt < NCH:
            compute_nid(nxt)
            if c >= 1:
                ocp[nxt % NB].wait()
            gcp[nxt % NB] = gather(nxt)
        gcp[cur].wait()
        ocp[cur] = pltpu.async_copy(
            rows[cur], neighs_out.at[pl.ds(base + c * CHUNK, CHUNK)],
            osems[cur])
    ocp[(NCH - 3) % NB].wait()
    ocp[(NCH - 2) % NB].wait()
    ocp[(NCH - 1) % NB].wait()

    pltpu.async_copy(proj_hbm.at[nidx_v], rows0.at[pl.ds(0, BPW)],
                     nsem).wait()
    pltpu.sync_copy(rows0.at[pl.ds(0, BPW)], nodes_out.at[pl.ds(nb, BPW)])


_sc_gather = pl.kernel(
    _sc_gather_body,
    out_type=[
        jax.ShapeDtypeStruct((T, D), jnp.float32),
        jax.ShapeDtypeStruct((B_ROWS, D), jnp.float32),
    ],
    mesh=plsc.VectorSubcoreMesh(core_axis_name="c", subcore_axis_name="s"),
    compiler_params=pltpu.CompilerParams(needs_layout_passes=False),
    scratch_types=[
        pltpu.VMEM((U_SIZE,), jnp.int32),
        pltpu.VMEM((TPW,), jnp.int32),
        pltpu.VMEM((NCH, CHUNK), jnp.int32),
        pltpu.VMEM((BPW,), jnp.int32),
        pltpu.VMEM((CHUNK, D), jnp.float32),
        pltpu.VMEM((CHUNK, D), jnp.float32),
        pltpu.VMEM((CHUNK, D), jnp.float32),
        pltpu.SemaphoreType.DMA,
        pltpu.SemaphoreType.DMA,
        pltpu.SemaphoreType.DMA,
        pltpu.SemaphoreType.DMA,
        pltpu.SemaphoreType.DMA,
        pltpu.SemaphoreType.DMA,
        pltpu.SemaphoreType.DMA,
    ],
)


def kernel(nodes, unique_nodes_list, samp_neighs_t, seq_length, features,
           W_in, b_in):
    nodes_i = nodes.astype(jnp.int32)
    uni_i = unique_nodes_list.astype(jnp.int32)
    samp_i = samp_neighs_t.astype(jnp.int32).reshape(-1)

    proj = pl.pallas_call(
        _proj_body,
        grid=(pl.cdiv(N_NODES, BM),),
        in_specs=[
            pl.BlockSpec((FEAT, BM), lambda i: (0, i)),
            pl.BlockSpec((FEAT, D), lambda i: (0, 0)),
            pl.BlockSpec((1, D), lambda i: (0, 0)),
        ],
        out_specs=pl.BlockSpec((BM, D), lambda i: (i, 0)),
        out_shape=jax.ShapeDtypeStruct((N_NODES, D), jnp.float32),
        compiler_params=pltpu.CompilerParams(
            fuse_transposed_lhs_in_matmul=True),
    )(features.T, W_in, b_in.reshape(1, D))

    mask = pl.pallas_call(
        _mask_body,
        out_shape=jax.ShapeDtypeStruct((B_ROWS, S), jnp.bool_),
    )(seq_length.astype(jnp.int32).reshape(B_ROWS, 1))

    neighs_flat, nodes_emb = _sc_gather(proj, uni_i, samp_i, nodes_i)
    return (nodes_emb, neighs_flat.reshape(B_ROWS, S, D), samp_neighs_t,
            mask)


# speedup vs baseline: 1.0242x; 1.0163x over previous
import jax
import jax.numpy as jnp
from jax import lax
from jax.experimental import pallas as pl
from jax.experimental.pallas import tpu as pltpu, tpu_sc as plsc

N_NODES = 10000
FEAT = 1433
D = 512
B_ROWS = 1024
S = 48
U_SIZE = 16384
T = B_ROWS * S

NC, NS = 2, 16
NW = NC * NS
TPW = T // NW
BPW = B_ROWS // NW
CHUNK = 64
NCH = TPW // CHUNK

BM = 2048


def _proj_body(ft_ref, w_ref, b_ref, o_ref):
    o_ref[...] = (
        lax.dot_general(ft_ref[...].astype(jnp.bfloat16),
                        w_ref[...].astype(jnp.bfloat16),
                        (((0,), (0,)), ((), ())),
                        preferred_element_type=jnp.float32)
        + b_ref[...]
    )


def _mask_body(seq_ref, o_ref):
    col = lax.broadcasted_iota(jnp.int32, (B_ROWS, S), 1) + 1
    o_ref[...] = col > seq_ref[...]


def _sc_gather_body(proj_hbm, uni_hbm, samp_hbm, nodes_hbm,
                    neighs_out, nodes_out,
                    samp_v, nid_v, nidx_v, rows0, rows1, rows2,
                    gsem0, gsem1, gsem2, osem0, osem1, osem2,
                    isem0, isem1, isem2, nsem):
    sid = lax.axis_index("s")
    wid = lax.axis_index("c") * NS + sid
    base = wid * TPW
    nb = wid * BPW

    samp_cp = pltpu.async_copy(samp_hbm.at[pl.ds(base, TPW)], samp_v, gsem0)
    nidx_cp = pltpu.async_copy(nodes_hbm.at[pl.ds(nb, BPW)], nidx_v, nsem)
    samp_cp.wait()
    nidx_cp.wait()

    gsems = (gsem0, gsem1, gsem2)
    osems = (osem0, osem1, osem2)
    isems = (isem0, isem1, isem2)
    rows = (rows0, rows1, rows2)
    NB = 3

    def idx_gather(c):
        return pltpu.async_copy(
            uni_hbm.at[samp_v.at[pl.ds(c * CHUNK, CHUNK)]], nid_v.at[c],
            isems[c % NB])

    def row_gather(c):
        return pltpu.async_copy(proj_hbm.at[nid_v.at[c]], rows[c % NB],
                                gsems[c % NB])

    icp = [None, None, None]
    gcp = [None, None, None]
    ocp = [None, None, None]
    icp[0] = idx_gather(0)
    icp[1] = idx_gather(1)
    icp[0].wait()
    gcp[0] = row_gather(0)
    icp[2] = idx_gather(2)
    icp[1].wait()
    gcp[1] = row_gather(1)
    for c in range(NCH):
        cur = c % NB
        nxt = c + NB - 1
        if nxt < NCH:
            if nxt + 1 < NCH:
                icp[(nxt + 1) % NB] = idx_gather(nxt + 1)
            if c >= 1:
                ocp[nxt % NB].wait()
            icp[nxt % NB].wait()
            gcp[nxt % NB] = row_gather(nxt)
        gcp[cur].wait()
        ocp[cur] = pltpu.async_copy(
            rows[cur], neighs_out.at[pl.ds(base + c * CHUNK, CHUNK)],
            osems[cur])
    ocp[(NCH - 3) % NB].wait()
    ocp[(NCH - 2) % NB].wait()
    ocp[(NCH - 1) % NB].wait()

    pltpu.async_copy(proj_hbm.at[nidx_v], rows0.at[pl.ds(0, BPW)],
                     nsem).wait()
    pltpu.sync_copy(rows0.at[pl.ds(0, BPW)], nodes_out.at[pl.ds(nb, BPW)])


_sc_gather = pl.kernel(
    _sc_gather_body,
    out_type=[
        jax.ShapeDtypeStruct((T, D), jnp.float32),
        jax.ShapeDtypeStruct((B_ROWS, D), jnp.float32),
    ],
    mesh=plsc.VectorSubcoreMesh(core_axis_name="c", subcore_axis_name="s"),
    scratch_types=[
        pltpu.VMEM((TPW,), jnp.int32),
        pltpu.VMEM((NCH, CHUNK), jnp.int32),
        pltpu.VMEM((BPW,), jnp.int32),
        pltpu.VMEM((CHUNK, D), jnp.float32),
        pltpu.VMEM((CHUNK, D), jnp.float32),
        pltpu.VMEM((CHUNK, D), jnp.float32),
        pltpu.SemaphoreType.DMA,
        pltpu.SemaphoreType.DMA,
        pltpu.SemaphoreType.DMA,
        pltpu.SemaphoreType.DMA,
        pltpu.SemaphoreType.DMA,
        pltpu.SemaphoreType.DMA,
        pltpu.SemaphoreType.DMA,
        pltpu.SemaphoreType.DMA,
        pltpu.SemaphoreType.DMA,
        pltpu.SemaphoreType.DMA,
    ],
)


def kernel(nodes, unique_nodes_list, samp_neighs_t, seq_length, features,
           W_in, b_in):
    nodes_i = nodes.astype(jnp.int32)
    uni_i = unique_nodes_list.astype(jnp.int32)
    samp_i = samp_neighs_t.astype(jnp.int32).reshape(-1)

    proj = pl.pallas_call(
        _proj_body,
        grid=(pl.cdiv(N_NODES, BM),),
        in_specs=[
            pl.BlockSpec((FEAT, BM), lambda i: (0, i)),
            pl.BlockSpec((FEAT, D), lambda i: (0, 0)),
            pl.BlockSpec((1, D), lambda i: (0, 0)),
        ],
        out_specs=pl.BlockSpec((BM, D), lambda i: (i, 0)),
        out_shape=jax.ShapeDtypeStruct((N_NODES, D), jnp.float32),
        compiler_params=pltpu.CompilerParams(
            fuse_transposed_lhs_in_matmul=True),
    )(features.T, W_in, b_in.reshape(1, D))

    mask = pl.pallas_call(
        _mask_body,
        out_shape=jax.ShapeDtypeStruct((B_ROWS, S), jnp.bool_),
    )(seq_length.astype(jnp.int32).reshape(B_ROWS, 1))

    neighs_flat, nodes_emb = _sc_gather(proj, uni_i, samp_i, nodes_i)
    return (nodes_emb, neighs_flat.reshape(B_ROWS, S, D), samp_neighs_t,
            mask)
